# Initial kernel scaffold; baseline (speedup 1.0000x reference)
import jax
import jax.numpy as jnp
from jax.experimental import pallas as pl


def kernel(x_cont, x_binary, x_cat, W_cont, b_cont, binary_tables, cat_tables):
    raise NotImplementedError("write your pallas kernel here")


# speedup vs baseline: 2.2996x; 2.2996x over previous
import functools

import jax
import jax.numpy as jnp
from jax import lax
from jax.experimental import pallas as pl
from jax.experimental.pallas import tpu as pltpu
from jax.experimental.pallas import tpu_sc as plsc

B = 16384
N_CONT = 13
N_BINARY = 26
N_CAT = 26
VOCAB = 100000
D_F = 32
N_FEAT = N_CONT + N_BINARY + N_CAT

NC, NS = 2, 16
NW = NC * NS
TOTAL_IDX = B * N_CAT
IDX_PER_W = TOTAL_IDX // NW
CHUNK = 1664

_MM_ROWS = 2048
_K = N_CONT + N_BINARY + 1
_CB_COLS = (N_CONT + N_BINARY) * D_F


def _gather_body(table_hbm, idx_hbm, out_hbm, idx_v, rows_v, sem):
    wid = lax.axis_index("s") * NC + lax.axis_index("c")
    base = wid * IDX_PER_W

    @pl.loop(0, IDX_PER_W, step=CHUNK)
    def _(off):
        pltpu.sync_copy(idx_hbm.at[pl.ds(base + off, CHUNK)], idx_v)
        pltpu.async_copy(table_hbm.at[idx_v], rows_v, sem).wait()
        pltpu.sync_copy(rows_v, out_hbm.at[pl.ds(base + off, CHUNK)])


def _cat_gather(table_flat, idx_flat):
    mesh = plsc.VectorSubcoreMesh(core_axis_name="c", subcore_axis_name="s")
    k = pl.kernel(
        _gather_body,
        out_type=jax.ShapeDtypeStruct((TOTAL_IDX, D_F), jnp.float32),
        mesh=mesh,
        scratch_types=[
            pltpu.VMEM((CHUNK,), jnp.int32),
            pltpu.VMEM((CHUNK, D_F), jnp.float32),
            pltpu.SemaphoreType.DMA,
        ],
        compiler_params=pltpu.CompilerParams(use_tc_tiling_on_sc=False),
    )
    return k(table_flat, idx_flat)


def _mm_body(x_ref, p_ref, o_ref):
    o_ref[...] = jnp.dot(
        x_ref[...], p_ref[...], preferred_element_type=jnp.float32
    )


def _cont_binary(x_aug, proj):
    return pl.pallas_call(
        _mm_body,
        grid=(B // _MM_ROWS,),
        in_specs=[
            pl.BlockSpec((_MM_ROWS, _K), lambda i: (i, 0)),
            pl.BlockSpec((_K, _CB_COLS), lambda i: (0, 0)),
        ],
        out_specs=pl.BlockSpec((_MM_ROWS, _CB_COLS), lambda i: (i, 0)),
        out_shape=jax.ShapeDtypeStruct((B, _CB_COLS), jnp.float32),
    )(x_aug, proj)


def kernel(x_cont, x_binary, x_cat, W_cont, b_cont, binary_tables, cat_tables):
    idx_flat = (
        x_cat.astype(jnp.int32)
        + (jnp.arange(N_CAT, dtype=jnp.int32) * VOCAB)[None, :]
    ).reshape(TOTAL_IDX)
    table_flat = cat_tables.reshape(N_CAT * VOCAB, D_F)

    xb = x_binary.astype(jnp.float32)
    ones = jnp.ones((B, 1), jnp.float32)
    x_aug = jnp.concatenate([x_cont, xb, ones], axis=1)

    t0 = binary_tables[:, 0, :]
    dt = binary_tables[:, 1, :] - t0
    w = W_cont[0]
    p_cont = (
        jnp.eye(N_CONT, dtype=jnp.float32)[:, :, None] * w[None, None, :]
    ).reshape(N_CONT, N_CONT * D_F)
    p_bin = (
        jnp.eye(N_BINARY, dtype=jnp.float32)[:, :, None] * dt[None, :, :]
    ).reshape(N_BINARY, N_BINARY * D_F)
    p_bias = jnp.concatenate(
        [jnp.tile(b_cont, N_CONT), t0.reshape(-1)]
    )[None, :]
    zeros_tr = jnp.zeros((N_CONT, N_BINARY * D_F), jnp.float32)
    zeros_bl = jnp.zeros((N_BINARY, N_CONT * D_F), jnp.float32)
    proj = jnp.concatenate(
        [
            jnp.concatenate([p_cont, zeros_tr], axis=1),
            jnp.concatenate([zeros_bl, p_bin], axis=1),
            p_bias,
        ],
        axis=0,
    )

    gathered = _cat_gather(table_flat, idx_flat)
    cb = _cont_binary(x_aug, proj)

    return jnp.concatenate(
        [cb.reshape(B, N_CONT + N_BINARY, D_F),
         gathered.reshape(B, N_CAT, D_F)],
        axis=1,
    )
